# Initial kernel scaffold; baseline (speedup 1.0000x reference)
import jax
import jax.numpy as jnp
from jax.experimental import pallas as pl


def kernel(src, seg, word_table, pos_table, seg_table, gamma, beta):
    raise NotImplementedError("write your pallas kernel here")


# speedup vs baseline: 1.3363x; 1.3363x over previous
import functools

import jax
import jax.numpy as jnp
from jax import lax
from jax.experimental import pallas as pl
from jax.experimental.pallas import tpu as pltpu
from jax.experimental.pallas import tpu_sc as plsc

NW = 32
CH = 128
LANES = 16


def _rsqrt_vec(v):
    i = lax.bitcast_convert_type(v, jnp.int32)
    i = jnp.int32(0x5F3759DF) - lax.shift_right_logical(i, 1)
    y = lax.bitcast_convert_type(i, jnp.float32)
    for _ in range(3):
        y = y * (1.5 - 0.5 * v * y * y)
    return y


def _make_sc_embed(N, D, n_chunks_w):
    nj = D // LANES
    per_w = n_chunks_w * CH
    mesh = plsc.VectorSubcoreMesh(core_axis_name="c", subcore_axis_name="s")

    @functools.partial(
        pl.kernel,
        mesh=mesh,
        out_type=jax.ShapeDtypeStruct((N, D), jnp.float32),
        compiler_params=pltpu.CompilerParams(use_tc_tiling_on_sc=False),
        scratch_types=[
            pltpu.VMEM((n_chunks_w, CH), jnp.int32),
            pltpu.VMEM((n_chunks_w, CH), jnp.int32),
            pltpu.VMEM((CH, D), jnp.float32),
            pltpu.VMEM((CH, D), jnp.float32),
            pltpu.VMEM((CH, D), jnp.float32),
            pltpu.VMEM((CH, D), jnp.float32),
            pltpu.VMEM((D,), jnp.float32),
            pltpu.VMEM((D,), jnp.float32),
            pltpu.VMEM((32,), jnp.float32),
            pltpu.VMEM((32,), jnp.float32),
            pltpu.SemaphoreType.DMA,
            pltpu.SemaphoreType.DMA,
            pltpu.SemaphoreType.DMA,
            pltpu.SemaphoreType.DMA,
        ],
    )
    def sc_embed(src_hbm, q_hbm, word_hbm, posseg_hbm, gamma_hbm, beta_hbm,
                 out_hbm, idx_v, q_v, rows0, rows1, prows0, prows1,
                 gamma_v, beta_v, scr_s, scr_q, sg0, sg1, ss0, ss1):
        wid = lax.axis_index("s") * 2 + lax.axis_index("c")
        wbase = wid * per_w
        rows = (rows0, rows1)
        prows = (prows0, prows1)
        sg = (sg0, sg1)
        ss = (ss0, ss1)

        pltpu.sync_copy(src_hbm.at[wid], idx_v)
        pltpu.sync_copy(q_hbm.at[wid], q_v)
        pltpu.sync_copy(gamma_hbm, gamma_v)
        pltpu.sync_copy(beta_hbm, beta_v)

        g = [gamma_v[pl.ds(LANES * j, LANES)] for j in range(nj)]
        bt = [beta_v[pl.ds(LANES * j, LANES)] for j in range(nj)]

        def gather(c, b):
            pltpu.async_copy(word_hbm.at[idx_v.at[c]], rows[b], sg[b])
            pltpu.async_copy(posseg_hbm.at[q_v.at[c]], prows[b], sg[b])

        def wait_gather(c, b):
            pltpu.make_async_copy(word_hbm.at[idx_v.at[c]], rows[b],
                                  sg[b]).wait()
            pltpu.make_async_copy(posseg_hbm.at[q_v.at[c]], prows[b],
                                  sg[b]).wait()

        def scatter(c, b):
            pltpu.async_copy(rows[b], out_hbm.at[pl.ds(wbase + c * CH, CH)],
                             ss[b])

        def wait_scatter(b):
            pltpu.make_async_copy(rows[b], out_hbm.at[pl.ds(wbase, CH)],
                                  ss[b]).wait()

        def rotate_reduce(v, scr):
            for sh in (8, 4, 2, 1):
                scr[pl.ds(0, LANES)] = v
                scr[pl.ds(LANES, LANES)] = v
                v = v + scr[pl.ds(sh, LANES)]
            return v

        def compute(b):
            def rbody(r, carry):
                x = [rows[b][r, pl.ds(LANES * j, LANES)] +
                     prows[b][r, pl.ds(LANES * j, LANES)] for j in range(nj)]
                s = functools.reduce(lambda a, c_: a + c_, x)
                s2 = functools.reduce(lambda a, c_: a + c_,
                                      [xj * xj for xj in x])
                totv = rotate_reduce(s, scr_s)
                tot2v = rotate_reduce(s2, scr_q)
                meanv = totv * (1.0 / D)
                varv = tot2v * (1.0 / D) - meanv * meanv + 1e-12
                rstd = _rsqrt_vec(varv)
                for j in range(nj):
                    rows[b][r, pl.ds(LANES * j, LANES)] = (
                        (x[j] - meanv) * rstd * g[j] + bt[j])
                return carry

            lax.fori_loop(0, CH, rbody, 0)

        gather(0, 0)

        def gbody(gi, carry):
            for b in range(2):
                c = 2 * gi + b

                @pl.when(c >= 1)
                def _w():
                    wait_scatter(1 - b)

                @pl.when(c + 1 < n_chunks_w)
                def _g():
                    gather(c + 1, 1 - b)

                wait_gather(c, b)
                compute(b)
                scatter(c, b)
            return carry

        lax.fori_loop(0, n_chunks_w // 2, gbody, 0)
        wait_scatter(1)

    return sc_embed


def kernel(src, seg, word_table, pos_table, seg_table, gamma, beta):
    B, L = src.shape
    _, D = word_table.shape
    S = seg_table.shape[0]
    N = B * L
    per_w = N // NW
    n_chunks_w = per_w // CH

    posseg = (pos_table[:L, None, :] + seg_table[None, :, :]).reshape(L * S, D)
    qidx = (jnp.arange(L, dtype=jnp.int32) * S)[None, :] + seg.astype(jnp.int32)
    src_w = src.astype(jnp.int32).reshape(NW, n_chunks_w, CH)
    q_w = qidx.reshape(NW, n_chunks_w, CH)

    out = _make_sc_embed(N, D, n_chunks_w)(
        src_w, q_w, word_table.astype(jnp.float32), posseg,
        gamma.astype(jnp.float32), beta.astype(jnp.float32))
    return out.reshape(B, L, D)


# speedup vs baseline: 1.8599x; 1.3918x over previous
import functools

import jax
import jax.numpy as jnp
from jax import lax
from jax.experimental import pallas as pl
from jax.experimental.pallas import tpu as pltpu
from jax.experimental.pallas import tpu_sc as plsc

NW = 32
CH = 128
LANES = 16
UNROLL = 4


def _rsqrt_vec(v):
    i = lax.bitcast_convert_type(v, jnp.int32)
    i = jnp.int32(0x5F3759DF) - lax.shift_right_logical(i, 1)
    y = lax.bitcast_convert_type(i, jnp.float32)
    for _ in range(3):
        y = y * (1.5 - 0.5 * v * y * y)
    return y


def _make_sc_embed(N, D, n_chunks_w):
    nj = D // LANES
    per_w = n_chunks_w * CH
    mesh = plsc.VectorSubcoreMesh(core_axis_name="c", subcore_axis_name="s")

    @functools.partial(
        pl.kernel,
        mesh=mesh,
        out_type=jax.ShapeDtypeStruct((N, D), jnp.float32),
        compiler_params=pltpu.CompilerParams(use_tc_tiling_on_sc=False),
        scratch_types=[
            pltpu.VMEM((n_chunks_w, CH), jnp.int32),
            pltpu.VMEM((n_chunks_w, CH), jnp.int32),
            pltpu.VMEM((CH, D), jnp.float32),
            pltpu.VMEM((CH, D), jnp.float32),
            pltpu.VMEM((CH, D), jnp.float32),
            pltpu.VMEM((CH, D), jnp.float32),
            pltpu.VMEM((D,), jnp.float32),
            pltpu.VMEM((D,), jnp.float32),
            pltpu.VMEM((UNROLL, 32), jnp.float32),
            pltpu.VMEM((UNROLL, 32), jnp.float32),
            pltpu.SemaphoreType.DMA,
            pltpu.SemaphoreType.DMA,
            pltpu.SemaphoreType.DMA,
            pltpu.SemaphoreType.DMA,
        ],
    )
    def sc_embed(src_hbm, q_hbm, word_hbm, posseg_hbm, gamma_hbm, beta_hbm,
                 out_hbm, idx_v, q_v, rows0, rows1, prows0, prows1,
                 gamma_v, beta_v, scr_s, scr_q, sg0, sg1, ss0, ss1):
        wid = lax.axis_index("s") * 2 + lax.axis_index("c")
        wbase = wid * per_w
        rows = (rows0, rows1)
        prows = (prows0, prows1)
        sg = (sg0, sg1)
        ss = (ss0, ss1)

        pltpu.sync_copy(src_hbm.at[wid], idx_v)
        pltpu.sync_copy(q_hbm.at[wid], q_v)
        pltpu.sync_copy(gamma_hbm, gamma_v)
        pltpu.sync_copy(beta_hbm, beta_v)

        g = [gamma_v[pl.ds(LANES * j, LANES)] for j in range(nj)]
        bt = [beta_v[pl.ds(LANES * j, LANES)] for j in range(nj)]

        def gather(c, b):
            pltpu.async_copy(word_hbm.at[idx_v.at[c]], rows[b], sg[b])
            pltpu.async_copy(posseg_hbm.at[q_v.at[c]], prows[b], sg[b])

        def wait_gather(c, b):
            pltpu.make_async_copy(word_hbm.at[idx_v.at[c]], rows[b],
                                  sg[b]).wait()
            pltpu.make_async_copy(posseg_hbm.at[q_v.at[c]], prows[b],
                                  sg[b]).wait()

        def scatter(c, b):
            pltpu.async_copy(rows[b], out_hbm.at[pl.ds(wbase + c * CH, CH)],
                             ss[b])

        def wait_scatter(b):
            pltpu.make_async_copy(rows[b], out_hbm.at[pl.ds(wbase, CH)],
                                  ss[b]).wait()

        def rotate_step(vs, scrs, us, sh):
            for v, scr in zip(vs, scrs):
                scr[pl.ds(0, LANES)] = v
                scr[pl.ds(LANES, LANES)] = v
            return [v + scr[pl.ds(sh, LANES)] for v, scr in zip(vs, scrs)]

        def compute(b):
            def rbody(it, carry):
                r0 = it * UNROLL
                xs, ss, qs = [], [], []
                for u in range(UNROLL):
                    r = r0 + u
                    x = [rows[b][r, pl.ds(LANES * j, LANES)] +
                         prows[b][r, pl.ds(LANES * j, LANES)]
                         for j in range(nj)]
                    xs.append(x)
                    ss.append(functools.reduce(lambda a, c_: a + c_, x))
                    qs.append(functools.reduce(lambda a, c_: a + c_,
                                               [xj * xj for xj in x]))
                sscr = [scr_s.at[u] for u in range(UNROLL)]
                qscr = [scr_q.at[u] for u in range(UNROLL)]
                for sh in (8, 4, 2, 1):
                    ss = rotate_step(ss, sscr, range(UNROLL), sh)
                    qs = rotate_step(qs, qscr, range(UNROLL), sh)
                for u in range(UNROLL):
                    meanv = ss[u] * (1.0 / D)
                    varv = qs[u] * (1.0 / D) - meanv * meanv + 1e-12
                    rstd = _rsqrt_vec(varv)
                    r = r0 + u
                    for j in range(nj):
                        rows[b][r, pl.ds(LANES * j, LANES)] = (
                            (xs[u][j] - meanv) * rstd * g[j] + bt[j])
                return carry

            lax.fori_loop(0, CH // UNROLL, rbody, 0)

        gather(0, 0)

        def gbody(gi, carry):
            for b in range(2):
                c = 2 * gi + b

                @pl.when(c >= 1)
                def _w():
                    wait_scatter(1 - b)

                @pl.when(c + 1 < n_chunks_w)
                def _g():
                    gather(c + 1, 1 - b)

                wait_gather(c, b)
                compute(b)
                scatter(c, b)
            return carry

        lax.fori_loop(0, n_chunks_w // 2, gbody, 0)
        wait_scatter(1)

    return sc_embed


def kernel(src, seg, word_table, pos_table, seg_table, gamma, beta):
    B, L = src.shape
    _, D = word_table.shape
    S = seg_table.shape[0]
    N = B * L
    per_w = N // NW
    n_chunks_w = per_w // CH

    posseg = (pos_table[:L, None, :] + seg_table[None, :, :]).reshape(L * S, D)
    qidx = (jnp.arange(L, dtype=jnp.int32) * S)[None, :] + seg.astype(jnp.int32)
    src_w = src.astype(jnp.int32).reshape(NW, n_chunks_w, CH)
    q_w = qidx.reshape(NW, n_chunks_w, CH)

    out = _make_sc_embed(N, D, n_chunks_w)(
        src_w, q_w, word_table.astype(jnp.float32), posseg,
        gamma.astype(jnp.float32), beta.astype(jnp.float32))
    return out.reshape(B, L, D)


# speedup vs baseline: 1.8661x; 1.0034x over previous
import functools

import jax
import jax.numpy as jnp
from jax import lax
from jax.experimental import pallas as pl
from jax.experimental.pallas import tpu as pltpu
from jax.experimental.pallas import tpu_sc as plsc

NW = 32
IB = 128
CH = 256
LANES = 16
UNROLL = 4


def _rsqrt_vec(v):
    i = lax.bitcast_convert_type(v, jnp.int32)
    i = jnp.int32(0x5F3759DF) - lax.shift_right_logical(i, 1)
    y = lax.bitcast_convert_type(i, jnp.float32)
    for _ in range(3):
        y = y * (1.5 - 0.5 * v * y * y)
    return y


def _make_sc_embed(B, L, D, PS):
    nj = D // LANES
    N = B * L
    per_w = N // NW
    n_chunks_w = per_w // CH
    nib = per_w // IB
    chunks_per_b = L // CH
    mesh = plsc.VectorSubcoreMesh(core_axis_name="c", subcore_axis_name="s")

    @functools.partial(
        pl.kernel,
        mesh=mesh,
        out_type=jax.ShapeDtypeStruct((B, L, D), jnp.float32),
        compiler_params=pltpu.CompilerParams(use_tc_tiling_on_sc=False),
        scratch_types=[
            pltpu.VMEM((nib, IB), jnp.int32),
            pltpu.VMEM((nib, IB), jnp.int32),
            pltpu.VMEM((CH, D), jnp.float32),
            pltpu.VMEM((CH, D), jnp.float32),
            pltpu.VMEM((CH, D), jnp.float32),
            pltpu.VMEM((CH, D), jnp.float32),
            pltpu.VMEM((D,), jnp.float32),
            pltpu.VMEM((D,), jnp.float32),
            pltpu.VMEM((UNROLL, 32), jnp.float32),
            pltpu.VMEM((UNROLL, 32), jnp.float32),
            pltpu.SemaphoreType.DMA,
            pltpu.SemaphoreType.DMA,
            pltpu.SemaphoreType.DMA,
            pltpu.SemaphoreType.DMA,
        ],
    )
    def sc_embed(src_hbm, q_hbm, word_hbm, posseg_hbm, gamma_hbm, beta_hbm,
                 out_hbm, idx_v, q_v, rows0, rows1, prows0, prows1,
                 gamma_v, beta_v, scr_s, scr_q, sg0, sg1, ss0, ss1):
        sid = lax.axis_index("s")
        wid = sid * 2 + lax.axis_index("c")
        wb0 = wid * (per_w // L)
        rows = (rows0, rows1)
        prows = (prows0, prows1)
        sg = (sg0, sg1)
        ss = (ss0, ss1)

        pltpu.sync_copy(src_hbm.at[wid], idx_v)
        pltpu.sync_copy(q_hbm.at[wid], q_v)
        pltpu.sync_copy(gamma_hbm, gamma_v)
        pltpu.sync_copy(beta_hbm, beta_v)

        g = [gamma_v[pl.ds(LANES * j, LANES)] for j in range(nj)]
        bt = [beta_v[pl.ds(LANES * j, LANES)] for j in range(nj)]

        def dmas(c, b):
            out = []
            for k in range(CH // IB):
                dst = rows[b].at[pl.ds(k * IB, IB)]
                out.append((word_hbm.at[idx_v.at[c * (CH // IB) + k]], dst))
                pdst = prows[b].at[pl.ds(k * IB, IB)]
                out.append((posseg_hbm.at[q_v.at[c * (CH // IB) + k]], pdst))
            return out

        def gather(c, b):
            for s_, d_ in dmas(c, b):
                pltpu.async_copy(s_, d_, sg[b])

        def wait_gather(c, b):
            for s_, d_ in dmas(c, b):
                pltpu.make_async_copy(s_, d_, sg[b]).wait()

        def out_slot(c):
            bb = wb0 + c // chunks_per_b
            l0 = (c % chunks_per_b) * CH
            return out_hbm.at[bb, pl.ds(l0, CH)]

        def scatter(c, b):
            pltpu.async_copy(rows[b], out_slot(c), ss[b])

        def wait_scatter(c, b):
            pltpu.make_async_copy(rows[b], out_slot(c), ss[b]).wait()

        def rotate_step(vs, scrs, sh):
            for v, scr in zip(vs, scrs):
                scr[pl.ds(0, LANES)] = v
                scr[pl.ds(LANES, LANES)] = v
            return [v + scr[pl.ds(sh, LANES)] for v, scr in zip(vs, scrs)]

        def compute(b):
            def rbody(it, carry):
                r0 = it * UNROLL
                xs, ss_, qs = [], [], []
                for u in range(UNROLL):
                    r = r0 + u
                    x = [rows[b][r, pl.ds(LANES * j, LANES)] +
                         prows[b][r, pl.ds(LANES * j, LANES)]
                         for j in range(nj)]
                    xs.append(x)
                    ss_.append(functools.reduce(lambda a, c_: a + c_, x))
                    qs.append(functools.reduce(lambda a, c_: a + c_,
                                               [xj * xj for xj in x]))
                sscr = [scr_s.at[u] for u in range(UNROLL)]
                qscr = [scr_q.at[u] for u in range(UNROLL)]
                for sh in (8, 4, 2, 1):
                    ss_ = rotate_step(ss_, sscr, sh)
                    qs = rotate_step(qs, qscr, sh)
                for u in range(UNROLL):
                    meanv = ss_[u] * (1.0 / D)
                    varv = qs[u] * (1.0 / D) - meanv * meanv + 1e-12
                    rstd = _rsqrt_vec(varv)
                    r = r0 + u
                    for j in range(nj):
                        rows[b][r, pl.ds(LANES * j, LANES)] = (
                            (xs[u][j] - meanv) * rstd * g[j] + bt[j])
                return carry

            lax.fori_loop(0, CH // UNROLL, rbody, 0)

        gather(0, 0)

        def gbody(gi, carry):
            for b in range(2):
                c = 2 * gi + b

                @pl.when(c >= 1)
                def _w():
                    wait_scatter(c - 1, 1 - b)

                @pl.when(c + 1 < n_chunks_w)
                def _g():
                    gather(c + 1, 1 - b)

                wait_gather(c, b)
                compute(b)
                scatter(c, b)
            return carry

        lax.fori_loop(0, n_chunks_w // 2, gbody, 0)
        wait_scatter(n_chunks_w - 1, 1)

    return sc_embed


def kernel(src, seg, word_table, pos_table, seg_table, gamma, beta):
    B, L = src.shape
    _, D = word_table.shape
    S = seg_table.shape[0]
    N = B * L
    per_w = N // NW
    nib = per_w // IB

    posseg = (pos_table[:L, None, :] + seg_table[None, :, :]).reshape(L * S, D)
    qidx = (jnp.arange(L, dtype=jnp.int32) * S)[None, :] + seg.astype(jnp.int32)
    src_w = src.astype(jnp.int32).reshape(NW, nib, IB)
    q_w = qidx.reshape(NW, nib, IB)

    return _make_sc_embed(B, L, D, L * S)(
        src_w, q_w, word_table.astype(jnp.float32), posseg,
        gamma.astype(jnp.float32), beta.astype(jnp.float32))


# speedup vs baseline: 1.9064x; 1.0216x over previous
import functools

import jax
import jax.numpy as jnp
from jax import lax
from jax.experimental import pallas as pl
from jax.experimental.pallas import tpu as pltpu
from jax.experimental.pallas import tpu_sc as plsc

NW = 32
IB = 128
CH = 256
LANES = 16
UNROLL = 8


def _rsqrt_vec(v):
    i = lax.bitcast_convert_type(v, jnp.int32)
    i = jnp.int32(0x5F3759DF) - lax.shift_right_logical(i, 1)
    y = lax.bitcast_convert_type(i, jnp.float32)
    for _ in range(3):
        y = y * (1.5 - 0.5 * v * y * y)
    return y


def _make_sc_embed(B, L, D, PS):
    nj = D // LANES
    N = B * L
    per_w = N // NW
    n_chunks_w = per_w // CH
    nib = per_w // IB
    chunks_per_b = L // CH
    mesh = plsc.VectorSubcoreMesh(core_axis_name="c", subcore_axis_name="s")

    @functools.partial(
        pl.kernel,
        mesh=mesh,
        out_type=jax.ShapeDtypeStruct((B, L, D), jnp.float32),
        compiler_params=pltpu.CompilerParams(use_tc_tiling_on_sc=False),
        scratch_types=[
            pltpu.VMEM((nib, IB), jnp.int32),
            pltpu.VMEM((nib, IB), jnp.int32),
            pltpu.VMEM((CH, D), jnp.float32),
            pltpu.VMEM((CH, D), jnp.float32),
            pltpu.VMEM((CH, D), jnp.float32),
            pltpu.VMEM((CH, D), jnp.float32),
            pltpu.VMEM((D,), jnp.float32),
            pltpu.VMEM((D,), jnp.float32),
            pltpu.VMEM((UNROLL, 32), jnp.float32),
            pltpu.VMEM((UNROLL, 32), jnp.float32),
            pltpu.SemaphoreType.DMA,
            pltpu.SemaphoreType.DMA,
            pltpu.SemaphoreType.DMA,
            pltpu.SemaphoreType.DMA,
        ],
    )
    def sc_embed(src_hbm, q_hbm, word_hbm, posseg_hbm, gamma_hbm, beta_hbm,
                 out_hbm, idx_v, q_v, rows0, rows1, prows0, prows1,
                 gamma_v, beta_v, scr_s, scr_q, sg0, sg1, ss0, ss1):
        sid = lax.axis_index("s")
        wid = sid * 2 + lax.axis_index("c")
        wb0 = wid * (per_w // L)
        rows = (rows0, rows1)
        prows = (prows0, prows1)
        sg = (sg0, sg1)
        ss = (ss0, ss1)

        pltpu.sync_copy(src_hbm.at[wid], idx_v)
        pltpu.sync_copy(q_hbm.at[wid], q_v)
        pltpu.sync_copy(gamma_hbm, gamma_v)
        pltpu.sync_copy(beta_hbm, beta_v)

        g = [gamma_v[pl.ds(LANES * j, LANES)] for j in range(nj)]
        bt = [beta_v[pl.ds(LANES * j, LANES)] for j in range(nj)]

        def dmas(c, b):
            out = []
            for k in range(CH // IB):
                dst = rows[b].at[pl.ds(k * IB, IB)]
                out.append((word_hbm.at[idx_v.at[c * (CH // IB) + k]], dst))
                pdst = prows[b].at[pl.ds(k * IB, IB)]
                out.append((posseg_hbm.at[q_v.at[c * (CH // IB) + k]], pdst))
            return out

        def gather(c, b):
            for s_, d_ in dmas(c, b):
                pltpu.async_copy(s_, d_, sg[b])

        def wait_gather(c, b):
            for s_, d_ in dmas(c, b):
                pltpu.make_async_copy(s_, d_, sg[b]).wait()

        def out_slot(c):
            bb = wb0 + c // chunks_per_b
            l0 = (c % chunks_per_b) * CH
            return out_hbm.at[bb, pl.ds(l0, CH)]

        def scatter(c, b):
            pltpu.async_copy(rows[b], out_slot(c), ss[b])

        def wait_scatter(c, b):
            pltpu.make_async_copy(rows[b], out_slot(c), ss[b]).wait()

        def rotate_step(vs, scrs, sh):
            for v, scr in zip(vs, scrs):
                scr[pl.ds(0, LANES)] = v
                scr[pl.ds(LANES, LANES)] = v
            return [v + scr[pl.ds(sh, LANES)] for v, scr in zip(vs, scrs)]

        def compute(b):
            def rbody(it, carry):
                r0 = it * UNROLL
                xs, ss_, qs = [], [], []
                for u in range(UNROLL):
                    r = r0 + u
                    x = [rows[b][r, pl.ds(LANES * j, LANES)] +
                         prows[b][r, pl.ds(LANES * j, LANES)]
                         for j in range(nj)]
                    xs.append(x)
                    ss_.append(functools.reduce(lambda a, c_: a + c_, x))
                    qs.append(functools.reduce(lambda a, c_: a + c_,
                                               [xj * xj for xj in x]))
                sscr = [scr_s.at[u] for u in range(UNROLL)]
                qscr = [scr_q.at[u] for u in range(UNROLL)]
                for sh in (8, 4, 2, 1):
                    ss_ = rotate_step(ss_, sscr, sh)
                    qs = rotate_step(qs, qscr, sh)
                for u in range(UNROLL):
                    meanv = ss_[u] * (1.0 / D)
                    varv = qs[u] * (1.0 / D) - meanv * meanv + 1e-12
                    rstd = _rsqrt_vec(varv)
                    r = r0 + u
                    for j in range(nj):
                        rows[b][r, pl.ds(LANES * j, LANES)] = (
                            (xs[u][j] - meanv) * rstd * g[j] + bt[j])
                return carry

            lax.fori_loop(0, CH // UNROLL, rbody, 0)

        gather(0, 0)

        def gbody(gi, carry):
            for b in range(2):
                c = 2 * gi + b

                @pl.when(c >= 1)
                def _w():
                    wait_scatter(c - 1, 1 - b)

                @pl.when(c + 1 < n_chunks_w)
                def _g():
                    gather(c + 1, 1 - b)

                wait_gather(c, b)
                compute(b)
                scatter(c, b)
            return carry

        lax.fori_loop(0, n_chunks_w // 2, gbody, 0)
        wait_scatter(n_chunks_w - 1, 1)

    return sc_embed


def kernel(src, seg, word_table, pos_table, seg_table, gamma, beta):
    B, L = src.shape
    _, D = word_table.shape
    S = seg_table.shape[0]
    N = B * L
    per_w = N // NW
    nib = per_w // IB

    posseg = (pos_table[:L, None, :] + seg_table[None, :, :]).reshape(L * S, D)
    qidx = (jnp.arange(L, dtype=jnp.int32) * S)[None, :] + seg.astype(jnp.int32)
    src_w = src.astype(jnp.int32).reshape(NW, nib, IB)
    q_w = qidx.reshape(NW, nib, IB)

    return _make_sc_embed(B, L, D, L * S)(
        src_w, q_w, word_table.astype(jnp.float32), posseg,
        gamma.astype(jnp.float32), beta.astype(jnp.float32))


# speedup vs baseline: 2.0277x; 1.0636x over previous
import functools

import jax
import jax.numpy as jnp
from jax import lax
from jax.experimental import pallas as pl
from jax.experimental.pallas import tpu as pltpu
from jax.experimental.pallas import tpu_sc as plsc

NW = 32
IB = 128
CH = 256
LANES = 16
UNROLL = 8


def _rsqrt_vec(v):
    i = lax.bitcast_convert_type(v, jnp.int32)
    i = jnp.int32(0x5F3759DF) - lax.shift_right_logical(i, 1)
    y = lax.bitcast_convert_type(i, jnp.float32)
    for _ in range(3):
        y = y * (1.5 - 0.5 * v * y * y)
    return y


def _make_sc_embed(B, L, D, PS):
    nj = D // LANES
    N = B * L
    per_w = N // NW
    n_chunks_w = per_w // CH
    nib = per_w // IB
    chunks_per_b = L // CH
    mesh = plsc.VectorSubcoreMesh(core_axis_name="c", subcore_axis_name="s")

    @functools.partial(
        pl.kernel,
        mesh=mesh,
        out_type=jax.ShapeDtypeStruct((B, L, D), jnp.float32),
        compiler_params=pltpu.CompilerParams(use_tc_tiling_on_sc=False),
        scratch_types=[
            pltpu.VMEM((nib, IB), jnp.int32),
            pltpu.VMEM((nib, IB), jnp.int32),
            pltpu.VMEM((CH, D), jnp.float32),
            pltpu.VMEM((CH, D), jnp.float32),
            pltpu.VMEM((CH, D), jnp.float32),
            pltpu.VMEM((CH, D), jnp.float32),
            pltpu.VMEM((D,), jnp.float32),
            pltpu.VMEM((D,), jnp.float32),
            pltpu.VMEM((CH, 32), jnp.float32),
            pltpu.VMEM((CH, 32), jnp.float32),
            pltpu.SemaphoreType.DMA,
            pltpu.SemaphoreType.DMA,
            pltpu.SemaphoreType.DMA,
            pltpu.SemaphoreType.DMA,
        ],
    )
    def sc_embed(src_hbm, q_hbm, word_hbm, posseg_hbm, gamma_hbm, beta_hbm,
                 out_hbm, idx_v, q_v, rows0, rows1, prows0, prows1,
                 gamma_v, beta_v, scr_s, scr_q, sg0, sg1, ss0, ss1):
        sid = lax.axis_index("s")
        wid = sid * 2 + lax.axis_index("c")
        wb0 = wid * (per_w // L)
        rows = (rows0, rows1)
        prows = (prows0, prows1)
        sg = (sg0, sg1)
        ss = (ss0, ss1)

        pltpu.sync_copy(src_hbm.at[wid], idx_v)
        pltpu.sync_copy(q_hbm.at[wid], q_v)
        pltpu.sync_copy(gamma_hbm, gamma_v)
        pltpu.sync_copy(beta_hbm, beta_v)

        g = [gamma_v[pl.ds(LANES * j, LANES)] for j in range(nj)]
        bt = [beta_v[pl.ds(LANES * j, LANES)] for j in range(nj)]

        def dmas(c, b):
            out = []
            for k in range(CH // IB):
                dst = rows[b].at[pl.ds(k * IB, IB)]
                out.append((word_hbm.at[idx_v.at[c * (CH // IB) + k]], dst))
                pdst = prows[b].at[pl.ds(k * IB, IB)]
                out.append((posseg_hbm.at[q_v.at[c * (CH // IB) + k]], pdst))
            return out

        def gather(c, b):
            for s_, d_ in dmas(c, b):
                pltpu.async_copy(s_, d_, sg[b])

        def wait_gather(c, b):
            for s_, d_ in dmas(c, b):
                pltpu.make_async_copy(s_, d_, sg[b]).wait()

        def out_slot(c):
            bb = wb0 + c // chunks_per_b
            l0 = (c % chunks_per_b) * CH
            return out_hbm.at[bb, pl.ds(l0, CH)]

        def scatter(c, b):
            pltpu.async_copy(rows[b], out_slot(c), ss[b])

        def wait_scatter(c, b):
            pltpu.make_async_copy(rows[b], out_slot(c), ss[b]).wait()

        def rotate_chain(vs, scr):
            out = []
            for v in vs:
                for sh in (8, 4, 2, 1):
                    scr[pl.ds(0, LANES)] = v
                    scr[pl.ds(LANES, LANES)] = v
                    v = v + scr[pl.ds(sh, LANES)]
                out.append(v)
            return out

        def compute(b):
            @plsc.parallel_loop(0, CH, 1, unroll=UNROLL)
            def rbody(r):
                x = [rows[b][r, pl.ds(LANES * j, LANES)] +
                     prows[b][r, pl.ds(LANES * j, LANES)]
                     for j in range(nj)]
                s = functools.reduce(lambda a, c_: a + c_, x)
                q = functools.reduce(lambda a, c_: a + c_,
                                     [xj * xj for xj in x])
                (s,) = rotate_chain([s], scr_s.at[r])
                (q,) = rotate_chain([q], scr_q.at[r])
                meanv = s * (1.0 / D)
                varv = q * (1.0 / D) - meanv * meanv + 1e-12
                rstd = _rsqrt_vec(varv)
                for j in range(nj):
                    rows[b][r, pl.ds(LANES * j, LANES)] = (
                        (x[j] - meanv) * rstd * g[j] + bt[j])

        gather(0, 0)

        def gbody(gi, carry):
            for b in range(2):
                c = 2 * gi + b

                @pl.when(c >= 1)
                def _w():
                    wait_scatter(c - 1, 1 - b)

                @pl.when(c + 1 < n_chunks_w)
                def _g():
                    gather(c + 1, 1 - b)

                wait_gather(c, b)
                compute(b)
                scatter(c, b)
            return carry

        lax.fori_loop(0, n_chunks_w // 2, gbody, 0)
        wait_scatter(n_chunks_w - 1, 1)

    return sc_embed


def kernel(src, seg, word_table, pos_table, seg_table, gamma, beta):
    B, L = src.shape
    _, D = word_table.shape
    S = seg_table.shape[0]
    N = B * L
    per_w = N // NW
    nib = per_w // IB

    posseg = (pos_table[:L, None, :] + seg_table[None, :, :]).reshape(L * S, D)
    qidx = (jnp.arange(L, dtype=jnp.int32) * S)[None, :] + seg.astype(jnp.int32)
    src_w = src.astype(jnp.int32).reshape(NW, nib, IB)
    q_w = qidx.reshape(NW, nib, IB)

    return _make_sc_embed(B, L, D, L * S)(
        src_w, q_w, word_table.astype(jnp.float32), posseg,
        gamma.astype(jnp.float32), beta.astype(jnp.float32))
